# baseline (speedup 1.0000x reference)
import jax
import jax.numpy as jnp
from jax import lax
from jax.experimental import pallas as pl
from jax.experimental.pallas import tpu as pltpu
from jax.experimental.pallas import tpu_sc as plsc

N = 10000
E = 320000
D = 128
ROWS = E // 128
CHUNK = 832
TAIL = ROWS - 3 * CHUNK
BE = 2048


def _make_gather(R):
  G = R // 8
  TR = R % 8
  NPS = 624

  def body(sf, tf, es2d, et2d, sx, sy, sz, tx, ty, tz,
           xs_out, xt_out, rad_out,
           idxs_v, idxt_v, fbuf_a, fbuf_b, rad_buf,
           sxv_a, syv_a, szv_a, txv_a, tyv_a, tzv_a,
           sxv_b, syv_b, szv_b, txv_b, tyv_b, tzv_b,
           tbl_sh, sem_a, sem_b, csem_a, csem_b):
    c = lax.axis_index("c")
    s = lax.axis_index("s")

    def stage(tbl):
      pltpu.sync_copy(tbl.at[pl.ds(s * NPS, NPS)],
                      tbl_sh.at[pl.ds(s * NPS, NPS)])

      @pl.when(s == 15)
      def _():
        pltpu.sync_copy(tbl.at[pl.ds(N - 16, 16)],
                        tbl_sh.at[pl.ds(N - 16, 16)])

    @pl.when(c == 0)
    def _():
      stage(sf)

    @pl.when(c == 1)
    def _():
      stage(tf)

    plsc.subcore_barrier()

    def run_feats(keys, out):
      fbufs = (fbuf_a, fbuf_b)
      fsems = (sem_a, sem_b)

      def do_group(gr, nrows):
        pltpu.sync_copy(keys.at[pl.ds(gr, nrows)], idxs_v.at[pl.ds(0, nrows)])
        cp = pltpu.async_copy(tbl_sh.at[idxs_v.at[0]], fbufs[0], fsems[0])
        for j in range(nrows):
          if j + 1 < nrows:
            nxt = pltpu.async_copy(tbl_sh.at[idxs_v.at[j + 1]],
                                   fbufs[(j + 1) % 2], fsems[(j + 1) % 2])
          else:
            nxt = None
          cp.wait()
          pltpu.sync_copy(fbufs[j % 2], out.at[pl.ds((gr + j) * 128, 128)])
          cp = nxt

      base = G // 16
      rem = G % 16
      cnt = base + jnp.where(s < rem, 1, 0)
      gstart = s * base + jnp.minimum(s, rem)

      def loop_body(g, _):
        do_group((gstart + g) * 8, 8)
        return 0

      lax.fori_loop(0, cnt, loop_body, 0)

      if TR:
        @pl.when(s == 15)
        def _():
          do_group(G * 8, TR)

    @pl.when(c == 0)
    def _():
      run_feats(es2d, xs_out)

    @pl.when(c == 1)
    def _():
      run_feats(et2d, xt_out)

    banks = (
        (sxv_a, syv_a, szv_a, txv_a, tyv_a, tzv_a, csem_a),
        (sxv_b, syv_b, szv_b, txv_b, tyv_b, tzv_b, csem_b),
    )

    def rad_issue(j, bank):
      bsx, bsy, bsz, btx, bty, btz, sem = bank
      return [
          pltpu.async_copy(sx.at[idxs_v.at[j]], bsx, sem),
          pltpu.async_copy(sy.at[idxs_v.at[j]], bsy, sem),
          pltpu.async_copy(sz.at[idxs_v.at[j]], bsz, sem),
          pltpu.async_copy(tx.at[idxt_v.at[j]], btx, sem),
          pltpu.async_copy(ty.at[idxt_v.at[j]], bty, sem),
          pltpu.async_copy(tz.at[idxt_v.at[j]], btz, sem),
      ]

    def rad_finish(j, bank, cps):
      bsx, bsy, bsz, btx, bty, btz, sem = bank
      for cp in cps:
        cp.wait()
      for k in range(8):
        sl = pl.ds(k * 16, 16)
        dx = btx[sl] - bsx[sl]
        dy = bty[sl] - bsy[sl]
        dz = btz[sl] - bsz[sl]
        rad_buf[j, sl] = dx * dx + dy * dy + dz * dz

    def rad_group(gr, nrows):
      pltpu.sync_copy(es2d.at[pl.ds(gr, nrows)], idxs_v.at[pl.ds(0, nrows)])
      pltpu.sync_copy(et2d.at[pl.ds(gr, nrows)], idxt_v.at[pl.ds(0, nrows)])
      cps = rad_issue(0, banks[0])
      for j in range(nrows):
        nxt = rad_issue(j + 1, banks[(j + 1) % 2]) if j + 1 < nrows else None
        rad_finish(j, banks[j % 2], cps)
        cps = nxt
      pltpu.sync_copy(rad_buf.at[pl.ds(0, nrows)],
                      rad_out.at[pl.ds(gr, nrows)])

    G2 = G // 2
    base2 = G2 // 16
    rem2 = G2 % 16
    cnt2 = base2 + jnp.where(s < rem2, 1, 0)
    gstart2 = c * G2 + s * base2 + jnp.minimum(s, rem2)

    def rad_loop(g, _):
      rad_group((gstart2 + g) * 8, 8)
      return 0

    lax.fori_loop(0, cnt2, rad_loop, 0)

    if G % 2:
      @pl.when((c == 0) & (s == 0))
      def _():
        rad_group((G - 1) * 8, 8)

    if TR:
      @pl.when((c == 0) & (s == 15))
      def _():
        rad_group(G * 8, TR)

  mesh = plsc.VectorSubcoreMesh(core_axis_name="c", subcore_axis_name="s")
  cvec = lambda: pltpu.VMEM((128,), jnp.float32)
  return pl.kernel(
      body,
      out_type=(
          jax.ShapeDtypeStruct((R * 128, D), jnp.float32),
          jax.ShapeDtypeStruct((R * 128, D), jnp.float32),
          jax.ShapeDtypeStruct((R, 128), jnp.float32),
      ),
      mesh=mesh,
      scratch_types=(
          pltpu.VMEM((8, 128), jnp.int32),
          pltpu.VMEM((8, 128), jnp.int32),
          pltpu.VMEM((128, D), jnp.float32),
          pltpu.VMEM((128, D), jnp.float32),
          pltpu.VMEM((8, 128), jnp.float32),
          cvec(), cvec(), cvec(), cvec(), cvec(), cvec(),
          cvec(), cvec(), cvec(), cvec(), cvec(), cvec(),
          pltpu.VMEM_SHARED((N, D), jnp.float32),
          pltpu.SemaphoreType.DMA,
          pltpu.SemaphoreType.DMA,
          pltpu.SemaphoreType.DMA,
          pltpu.SemaphoreType.DMA,
      ),
  )


def _make_scatter(R):
  G = R // 8
  TR = R % 8
  NPS = 624

  def body(h2es, h2et, et2d, es2d, init0, init1,
           agg0, agg1, rows_a, rows_b, idx_v, acc_sh, sem_a, sem_b):
    c = lax.axis_index("c")
    s = lax.axis_index("s")

    def load_init(init):
      pltpu.sync_copy(init.at[pl.ds(s * NPS, NPS)],
                      acc_sh.at[pl.ds(s * NPS, NPS)])

      @pl.when(s == 15)
      def _():
        pltpu.sync_copy(init.at[pl.ds(N - 16, 16)],
                        acc_sh.at[pl.ds(N - 16, 16)])

    @pl.when(c == 0)
    def _():
      load_init(init0)

    @pl.when(c == 1)
    def _():
      load_init(init1)

    plsc.subcore_barrier()

    def run(data, keys):
      rbufs = (rows_a, rows_b)
      rsems = (sem_a, sem_b)

      def do_group(gr, nrows):
        pltpu.sync_copy(keys.at[pl.ds(gr, nrows)], idx_v.at[pl.ds(0, nrows)])
        cp = pltpu.async_copy(data.at[pl.ds(gr * 128, 128)], rbufs[0],
                              rsems[0])
        for j in range(nrows):
          if j + 1 < nrows:
            nxt = pltpu.async_copy(
                data.at[pl.ds((gr + j + 1) * 128, 128)],
                rbufs[(j + 1) % 2], rsems[(j + 1) % 2])
          else:
            nxt = None
          cp.wait()
          pltpu.sync_copy(rbufs[j % 2], acc_sh.at[idx_v.at[j]], add=True)
          cp = nxt

      base = G // 16
      rem = G % 16
      cnt = base + jnp.where(s < rem, 1, 0)
      gstart = s * base + jnp.minimum(s, rem)

      def loop_body(g, _):
        do_group((gstart + g) * 8, 8)
        return 0

      lax.fori_loop(0, cnt, loop_body, 0)

      if TR:
        @pl.when(s == 15)
        def _():
          do_group(G * 8, TR)

    @pl.when(c == 0)
    def _():
      run(h2es, et2d)

    @pl.when(c == 1)
    def _():
      run(h2et, es2d)

    plsc.subcore_barrier()

    def writeout(agg):
      pltpu.sync_copy(acc_sh.at[pl.ds(s * NPS, NPS)],
                      agg.at[pl.ds(s * NPS, NPS)])

      @pl.when(s == 15)
      def _():
        pltpu.sync_copy(acc_sh.at[pl.ds(N - 16, 16)],
                        agg.at[pl.ds(N - 16, 16)])

    @pl.when(c == 0)
    def _():
      writeout(agg0)

    @pl.when(c == 1)
    def _():
      writeout(agg1)

  mesh = plsc.VectorSubcoreMesh(core_axis_name="c", subcore_axis_name="s")
  return pl.kernel(
      body,
      out_type=(
          jax.ShapeDtypeStruct((N, D), jnp.float32),
          jax.ShapeDtypeStruct((N, D), jnp.float32),
      ),
      mesh=mesh,
      scratch_types=(
          pltpu.VMEM((128, D), jnp.float32),
          pltpu.VMEM((128, D), jnp.float32),
          pltpu.VMEM((8, 128), jnp.int32),
          pltpu.VMEM_SHARED((N, D), jnp.float32),
          pltpu.SemaphoreType.DMA,
          pltpu.SemaphoreType.DMA,
      ),
  )


def _edge_mlp_body(be, br, xs_ref, xt_ref, rad_ref, w1s_ref, w1t_ref, vb_ref,
                   bb_ref, w2es_ref, b2es_ref, w2et_ref, b2et_ref,
                   oes_ref, oet_ref):
  xs = xs_ref[...]
  xt = xt_ref[...]
  r_rows = rad_ref[0]
  qe = lax.broadcasted_iota(jnp.int32, (be, br), 0) // 128
  qq = lax.broadcasted_iota(jnp.int32, (be, br), 1)
  p = (qe == qq).astype(jnp.float32)
  s1 = jnp.dot(p, r_rows, preferred_element_type=jnp.float32)
  le = lax.broadcasted_iota(jnp.int32, (be, 128), 0) % 128
  ll = lax.broadcasted_iota(jnp.int32, (be, 128), 1)
  dmask = (le == ll).astype(jnp.float32)
  r_col = jnp.sum(s1 * dmask, axis=1, keepdims=True)

  pre = (jnp.dot(xs, w1s_ref[...], preferred_element_type=jnp.float32)
         + jnp.dot(xt, w1t_ref[...], preferred_element_type=jnp.float32))
  v = vb_ref[0:1, :]
  b = bb_ref[0:1, :]
  h1 = jnp.maximum(pre + r_col * v + b, 0.0)
  h1es = h1[:, :128]
  h1et = h1[:, 128:]
  h2es = jnp.maximum(
      jnp.dot(h1es, w2es_ref[...], preferred_element_type=jnp.float32)
      + b2es_ref[0:1, :], 0.0)
  h2et = jnp.maximum(
      jnp.dot(h1et, w2et_ref[...], preferred_element_type=jnp.float32)
      + b2et_ref[0:1, :], 0.0)
  oes_ref[...] = h2es
  oet_ref[...] = h2et


def _tc_edge_mlp(xs, xt, rad, w1s, w1t, vb, bb, w2es, b2es, w2et, b2et):
  e = xs.shape[0]
  be = min(BE, e)
  br = be // 128
  grid = (e // be,)
  full = lambda i: (0, 0)
  import functools
  return pl.pallas_call(
      functools.partial(_edge_mlp_body, be, br),
      grid=grid,
      in_specs=[
          pl.BlockSpec((be, D), lambda i: (i, 0)),
          pl.BlockSpec((be, D), lambda i: (i, 0)),
          pl.BlockSpec((1, br, 128), lambda i: (i, 0, 0)),
          pl.BlockSpec((D, 256), full),
          pl.BlockSpec((D, 256), full),
          pl.BlockSpec((8, 256), full),
          pl.BlockSpec((8, 256), full),
          pl.BlockSpec((D, D), full),
          pl.BlockSpec((8, D), full),
          pl.BlockSpec((D, D), full),
          pl.BlockSpec((8, D), full),
      ],
      out_specs=[
          pl.BlockSpec((be, D), lambda i: (i, 0)),
          pl.BlockSpec((be, D), lambda i: (i, 0)),
      ],
      out_shape=[
          jax.ShapeDtypeStruct((e, D), jnp.float32),
          jax.ShapeDtypeStruct((e, D), jnp.float32),
      ],
  )(xs, xt, rad.reshape(e // be, br, 128), w1s, w1t, vb, bb,
    w2es, b2es, w2et, b2et)


BN = 1000


def _node_mlp_body(tf_ref, sf_ref, a0_ref, a1_ref,
                   w1tf_ref, w1ta_ref, b1t_ref, w2t_ref, b2t_ref,
                   w1sf_ref, w1sa_ref, b1s_ref, w2s_ref, b2s_ref,
                   to_ref, so_ref):
  tf = tf_ref[...]
  sf = sf_ref[...]
  a0 = a0_ref[...]
  a1 = a1_ref[...]
  ht = jnp.maximum(
      jnp.dot(tf, w1tf_ref[...], preferred_element_type=jnp.float32)
      + jnp.dot(a0, w1ta_ref[...], preferred_element_type=jnp.float32)
      + b1t_ref[0:1, :], 0.0)
  to_ref[...] = tf + jnp.dot(ht, w2t_ref[...],
                             preferred_element_type=jnp.float32) + b2t_ref[0:1, :]
  hs = jnp.maximum(
      jnp.dot(sf, w1sf_ref[...], preferred_element_type=jnp.float32)
      + jnp.dot(a1, w1sa_ref[...], preferred_element_type=jnp.float32)
      + b1s_ref[0:1, :], 0.0)
  so_ref[...] = sf + jnp.dot(hs, w2s_ref[...],
                             preferred_element_type=jnp.float32) + b2s_ref[0:1, :]


def _tc_node_mlp(tf, sf, a0, a1, w1tf, w1ta, b1t, w2t, b2t,
                 w1sf, w1sa, b1s, w2s, b2s):
  grid = (N // BN,)
  full = lambda i: (0, 0)
  blk = lambda i: (i, 0)
  wspec = pl.BlockSpec((D, D), full)
  bspec = pl.BlockSpec((8, D), full)
  return pl.pallas_call(
      _node_mlp_body,
      grid=grid,
      in_specs=[
          pl.BlockSpec((BN, D), blk),
          pl.BlockSpec((BN, D), blk),
          pl.BlockSpec((BN, D), blk),
          pl.BlockSpec((BN, D), blk),
          wspec, wspec, bspec, wspec, bspec,
          wspec, wspec, bspec, wspec, bspec,
      ],
      out_specs=[
          pl.BlockSpec((BN, D), blk),
          pl.BlockSpec((BN, D), blk),
      ],
      out_shape=[
          jax.ShapeDtypeStruct((N, D), jnp.float32),
          jax.ShapeDtypeStruct((N, D), jnp.float32),
      ],
  )(tf, sf, a0, a1, w1tf, w1ta, b1t, w2t, b2t, w1sf, w1sa, b1s, w2s, b2s)


def kernel(src_node_feat, tgt_node_feat, src_node_coord, tgt_node_coord,
           edge_list, es_W1, es_b1, es_W2, es_b2, et_W1, et_b1, et_W2, et_b2,
           ns_W1, ns_b1, ns_W2, ns_b2, nt_W1, nt_b1, nt_W2, nt_b2):
  f32 = jnp.float32
  es = edge_list[0].astype(jnp.int32)
  et = edge_list[1].astype(jnp.int32)
  es2d = es.reshape(ROWS, 128)
  et2d = et.reshape(ROWS, 128)

  def coord_cols(c):
    return (c[:, 0], c[:, 1], c[:, 2])

  coords = coord_cols(src_node_coord) + coord_cols(tgt_node_coord)

  w1s = jnp.concatenate([es_W1[:, :D].T, et_W1[:, :D].T], axis=1)
  w1t = jnp.concatenate([es_W1[:, D:2 * D].T, et_W1[:, D:2 * D].T], axis=1)
  vb = jnp.tile(jnp.concatenate([es_W1[:, 2 * D], et_W1[:, 2 * D]])[None, :],
                (8, 1))
  bb = jnp.tile(jnp.concatenate([es_b1, et_b1])[None, :], (8, 1))
  t8 = lambda x: jnp.tile(x[None, :], (8, 1))
  mlp_w = (w1s, w1t, vb, bb, es_W2.T, t8(es_b2), et_W2.T, t8(et_b2))

  gather_main = _make_gather(CHUNK)
  gather_tail = _make_gather(TAIL)
  scatter_main = _make_scatter(CHUNK)
  scatter_tail = _make_scatter(TAIL)

  bounds = [(0, CHUNK, gather_main, scatter_main),
            (CHUNK, 2 * CHUNK, gather_main, scatter_main),
            (2 * CHUNK, 3 * CHUNK, gather_main, scatter_main),
            (3 * CHUNK, ROWS, gather_tail, scatter_tail)]

  gathered = []
  for lo, hi, gfn, _ in bounds:
    gathered.append(gfn(src_node_feat, tgt_node_feat,
                        es2d[lo:hi], et2d[lo:hi], *coords))

  agg0 = jnp.zeros((N, D), f32)
  agg1 = jnp.zeros((N, D), f32)
  for (lo, hi, _, sfn), (xs, xt, rad) in zip(bounds, gathered):
    h2es, h2et = _tc_edge_mlp(xs, xt, rad, *mlp_w)
    agg0, agg1 = sfn(h2es, h2et, et2d[lo:hi], es2d[lo:hi], agg0, agg1)

  tgt_out, src_out = _tc_node_mlp(
      tgt_node_feat, src_node_feat, agg0, agg1,
      nt_W1[:, :D].T, nt_W1[:, D:].T, t8(nt_b1), nt_W2.T, t8(nt_b2),
      ns_W1[:, :D].T, ns_W1[:, D:].T, t8(ns_b1), ns_W2.T, t8(ns_b2))
  return (tgt_out, src_out)


# speedup vs baseline: 1.0590x; 1.0590x over previous
import jax
import jax.numpy as jnp
from jax import lax
from jax.experimental import pallas as pl
from jax.experimental.pallas import tpu as pltpu
from jax.experimental.pallas import tpu_sc as plsc

N = 10000
E = 320000
D = 128
ROWS = E // 128
CHUNK = 1248
TAIL = ROWS - 2 * CHUNK
BE = 2048


def _make_gather(R):
  G = R // 8
  TR = R % 8
  NPS = 624

  def body(sf, tf, es2d, et2d, sx, sy, sz, tx, ty, tz,
           xs_out, xt_out, rad_out,
           idxs_v, idxt_v, fbuf_a, fbuf_b, rad_buf,
           sxv_a, syv_a, szv_a, txv_a, tyv_a, tzv_a,
           sxv_b, syv_b, szv_b, txv_b, tyv_b, tzv_b,
           tbl_sh, sem_a, sem_b, csem_a, csem_b):
    c = lax.axis_index("c")
    s = lax.axis_index("s")

    def stage(tbl):
      pltpu.sync_copy(tbl.at[pl.ds(s * NPS, NPS)],
                      tbl_sh.at[pl.ds(s * NPS, NPS)])

      @pl.when(s == 15)
      def _():
        pltpu.sync_copy(tbl.at[pl.ds(N - 16, 16)],
                        tbl_sh.at[pl.ds(N - 16, 16)])

    @pl.when(c == 0)
    def _():
      stage(sf)

    @pl.when(c == 1)
    def _():
      stage(tf)

    plsc.subcore_barrier()

    G2 = G // 2
    base2 = G2 // 16
    rem2 = G2 % 16
    cnt2 = base2 + jnp.where(s < rem2, 1, 0)
    rstart = c * G2 + s * base2 + jnp.minimum(s, rem2)
    if G % 2:
      cnt2 = cnt2 + jnp.where((c == 0) & (s == 15), 1, 0)

    def run_feats(keys, out):
      fbufs = (fbuf_a, fbuf_b)
      fsems = (sem_a, sem_b)

      def do_group(gr, nrows):
        pltpu.sync_copy(keys.at[pl.ds(gr, nrows)], idxs_v.at[pl.ds(0, nrows)])
        cp = pltpu.async_copy(tbl_sh.at[idxs_v.at[0]], fbufs[0], fsems[0])
        for j in range(nrows):
          if j + 1 < nrows:
            nxt = pltpu.async_copy(tbl_sh.at[idxs_v.at[j + 1]],
                                   fbufs[(j + 1) % 2], fsems[(j + 1) % 2])
          else:
            nxt = None
          cp.wait()
          pltpu.sync_copy(fbufs[j % 2], out.at[pl.ds((gr + j) * 128, 128)])
          cp = nxt

      base = G // 16
      rem = G % 16
      cnt = base + jnp.where(s < rem, 1, 0)
      gstart = s * base + jnp.minimum(s, rem)

      def loop_body(g, _):
        @pl.when(g < cnt)
        def _():
          do_group((gstart + g) * 8, 8)

        @pl.when(g < cnt2)
        def _():
          rad_group((rstart + g) * 8, 8)
        return 0

      lax.fori_loop(0, jnp.maximum(cnt, cnt2), loop_body, 0)

      if TR:
        @pl.when(s == 15)
        def _():
          do_group(G * 8, TR)

    banks = (
        (sxv_a, syv_a, szv_a, txv_a, tyv_a, tzv_a, csem_a),
        (sxv_b, syv_b, szv_b, txv_b, tyv_b, tzv_b, csem_b),
    )

    def rad_issue(j, bank):
      bsx, bsy, bsz, btx, bty, btz, sem = bank
      return [
          pltpu.async_copy(sx.at[idxs_v.at[j]], bsx, sem),
          pltpu.async_copy(sy.at[idxs_v.at[j]], bsy, sem),
          pltpu.async_copy(sz.at[idxs_v.at[j]], bsz, sem),
          pltpu.async_copy(tx.at[idxt_v.at[j]], btx, sem),
          pltpu.async_copy(ty.at[idxt_v.at[j]], bty, sem),
          pltpu.async_copy(tz.at[idxt_v.at[j]], btz, sem),
      ]

    def rad_finish(j, bank, cps):
      bsx, bsy, bsz, btx, bty, btz, sem = bank
      for cp in cps:
        cp.wait()
      for k in range(8):
        sl = pl.ds(k * 16, 16)
        dx = btx[sl] - bsx[sl]
        dy = bty[sl] - bsy[sl]
        dz = btz[sl] - bsz[sl]
        rad_buf[j, sl] = dx * dx + dy * dy + dz * dz

    def rad_group(gr, nrows):
      pltpu.sync_copy(es2d.at[pl.ds(gr, nrows)], idxs_v.at[pl.ds(0, nrows)])
      pltpu.sync_copy(et2d.at[pl.ds(gr, nrows)], idxt_v.at[pl.ds(0, nrows)])
      cps = rad_issue(0, banks[0])
      for j in range(nrows):
        nxt = rad_issue(j + 1, banks[(j + 1) % 2]) if j + 1 < nrows else None
        rad_finish(j, banks[j % 2], cps)
        cps = nxt
      pltpu.sync_copy(rad_buf.at[pl.ds(0, nrows)],
                      rad_out.at[pl.ds(gr, nrows)])

    @pl.when(c == 0)
    def _():
      run_feats(es2d, xs_out)

    @pl.when(c == 1)
    def _():
      run_feats(et2d, xt_out)

    if TR:
      @pl.when((c == 0) & (s == 15))
      def _():
        rad_group(G * 8, TR)

  mesh = plsc.VectorSubcoreMesh(core_axis_name="c", subcore_axis_name="s")
  cvec = lambda: pltpu.VMEM((128,), jnp.float32)
  return pl.kernel(
      body,
      out_type=(
          jax.ShapeDtypeStruct((R * 128, D), jnp.float32),
          jax.ShapeDtypeStruct((R * 128, D), jnp.float32),
          jax.ShapeDtypeStruct((R, 128), jnp.float32),
      ),
      mesh=mesh,
      scratch_types=(
          pltpu.VMEM((8, 128), jnp.int32),
          pltpu.VMEM((8, 128), jnp.int32),
          pltpu.VMEM((128, D), jnp.float32),
          pltpu.VMEM((128, D), jnp.float32),
          pltpu.VMEM((8, 128), jnp.float32),
          cvec(), cvec(), cvec(), cvec(), cvec(), cvec(),
          cvec(), cvec(), cvec(), cvec(), cvec(), cvec(),
          pltpu.VMEM_SHARED((N, D), jnp.float32),
          pltpu.SemaphoreType.DMA,
          pltpu.SemaphoreType.DMA,
          pltpu.SemaphoreType.DMA,
          pltpu.SemaphoreType.DMA,
      ),
  )


def _make_scatter(R):
  G = R // 8
  TR = R % 8
  NPS = 624

  def body(h2es, h2et, et2d, es2d, init0, init1,
           agg0, agg1, rows_a, rows_b, idx_v, acc_sh, sem_a, sem_b):
    c = lax.axis_index("c")
    s = lax.axis_index("s")

    def load_init(init):
      pltpu.sync_copy(init.at[pl.ds(s * NPS, NPS)],
                      acc_sh.at[pl.ds(s * NPS, NPS)])

      @pl.when(s == 15)
      def _():
        pltpu.sync_copy(init.at[pl.ds(N - 16, 16)],
                        acc_sh.at[pl.ds(N - 16, 16)])

    @pl.when(c == 0)
    def _():
      load_init(init0)

    @pl.when(c == 1)
    def _():
      load_init(init1)

    plsc.subcore_barrier()

    def run(data, keys):
      rbufs = (rows_a, rows_b)
      rsems = (sem_a, sem_b)

      def do_group(gr, nrows):
        pltpu.sync_copy(keys.at[pl.ds(gr, nrows)], idx_v.at[pl.ds(0, nrows)])
        cp = pltpu.async_copy(data.at[pl.ds(gr * 128, 128)], rbufs[0],
                              rsems[0])
        for j in range(nrows):
          if j + 1 < nrows:
            nxt = pltpu.async_copy(
                data.at[pl.ds((gr + j + 1) * 128, 128)],
                rbufs[(j + 1) % 2], rsems[(j + 1) % 2])
          else:
            nxt = None
          cp.wait()
          pltpu.sync_copy(rbufs[j % 2], acc_sh.at[idx_v.at[j]], add=True)
          cp = nxt

      base = G // 16
      rem = G % 16
      cnt = base + jnp.where(s < rem, 1, 0)
      gstart = s * base + jnp.minimum(s, rem)

      def loop_body(g, _):
        do_group((gstart + g) * 8, 8)
        return 0

      lax.fori_loop(0, cnt, loop_body, 0)

      if TR:
        @pl.when(s == 15)
        def _():
          do_group(G * 8, TR)

    @pl.when(c == 0)
    def _():
      run(h2es, et2d)

    @pl.when(c == 1)
    def _():
      run(h2et, es2d)

    plsc.subcore_barrier()

    def writeout(agg):
      pltpu.sync_copy(acc_sh.at[pl.ds(s * NPS, NPS)],
                      agg.at[pl.ds(s * NPS, NPS)])

      @pl.when(s == 15)
      def _():
        pltpu.sync_copy(acc_sh.at[pl.ds(N - 16, 16)],
                        agg.at[pl.ds(N - 16, 16)])

    @pl.when(c == 0)
    def _():
      writeout(agg0)

    @pl.when(c == 1)
    def _():
      writeout(agg1)

  mesh = plsc.VectorSubcoreMesh(core_axis_name="c", subcore_axis_name="s")
  return pl.kernel(
      body,
      out_type=(
          jax.ShapeDtypeStruct((N, D), jnp.float32),
          jax.ShapeDtypeStruct((N, D), jnp.float32),
      ),
      mesh=mesh,
      scratch_types=(
          pltpu.VMEM((128, D), jnp.float32),
          pltpu.VMEM((128, D), jnp.float32),
          pltpu.VMEM((8, 128), jnp.int32),
          pltpu.VMEM_SHARED((N, D), jnp.float32),
          pltpu.SemaphoreType.DMA,
          pltpu.SemaphoreType.DMA,
      ),
  )


def _edge_mlp_body(be, br, xs_ref, xt_ref, rad_ref, w1s_ref, w1t_ref, vb_ref,
                   bb_ref, w2es_ref, b2es_ref, w2et_ref, b2et_ref,
                   oes_ref, oet_ref):
  xs = xs_ref[...]
  xt = xt_ref[...]
  r_rows = rad_ref[0]
  qe = lax.broadcasted_iota(jnp.int32, (be, br), 0) // 128
  qq = lax.broadcasted_iota(jnp.int32, (be, br), 1)
  p = (qe == qq).astype(jnp.float32)
  s1 = jnp.dot(p, r_rows, preferred_element_type=jnp.float32)
  le = lax.broadcasted_iota(jnp.int32, (be, 128), 0) % 128
  ll = lax.broadcasted_iota(jnp.int32, (be, 128), 1)
  dmask = (le == ll).astype(jnp.float32)
  r_col = jnp.sum(s1 * dmask, axis=1, keepdims=True)

  pre = (jnp.dot(xs, w1s_ref[...], preferred_element_type=jnp.float32)
         + jnp.dot(xt, w1t_ref[...], preferred_element_type=jnp.float32))
  v = vb_ref[0:1, :]
  b = bb_ref[0:1, :]
  h1 = jnp.maximum(pre + r_col * v + b, 0.0)
  h1es = h1[:, :128]
  h1et = h1[:, 128:]
  h2es = jnp.maximum(
      jnp.dot(h1es, w2es_ref[...], preferred_element_type=jnp.float32)
      + b2es_ref[0:1, :], 0.0)
  h2et = jnp.maximum(
      jnp.dot(h1et, w2et_ref[...], preferred_element_type=jnp.float32)
      + b2et_ref[0:1, :], 0.0)
  oes_ref[...] = h2es
  oet_ref[...] = h2et


def _tc_edge_mlp(xs, xt, rad, w1s, w1t, vb, bb, w2es, b2es, w2et, b2et):
  e = xs.shape[0]
  be = min(BE, e)
  br = be // 128
  grid = (e // be,)
  full = lambda i: (0, 0)
  import functools
  return pl.pallas_call(
      functools.partial(_edge_mlp_body, be, br),
      grid=grid,
      in_specs=[
          pl.BlockSpec((be, D), lambda i: (i, 0)),
          pl.BlockSpec((be, D), lambda i: (i, 0)),
          pl.BlockSpec((1, br, 128), lambda i: (i, 0, 0)),
          pl.BlockSpec((D, 256), full),
          pl.BlockSpec((D, 256), full),
          pl.BlockSpec((8, 256), full),
          pl.BlockSpec((8, 256), full),
          pl.BlockSpec((D, D), full),
          pl.BlockSpec((8, D), full),
          pl.BlockSpec((D, D), full),
          pl.BlockSpec((8, D), full),
      ],
      out_specs=[
          pl.BlockSpec((be, D), lambda i: (i, 0)),
          pl.BlockSpec((be, D), lambda i: (i, 0)),
      ],
      out_shape=[
          jax.ShapeDtypeStruct((e, D), jnp.float32),
          jax.ShapeDtypeStruct((e, D), jnp.float32),
      ],
  )(xs, xt, rad.reshape(e // be, br, 128), w1s, w1t, vb, bb,
    w2es, b2es, w2et, b2et)


BN = 1000


def _node_mlp_body(tf_ref, sf_ref, a0_ref, a1_ref,
                   w1tf_ref, w1ta_ref, b1t_ref, w2t_ref, b2t_ref,
                   w1sf_ref, w1sa_ref, b1s_ref, w2s_ref, b2s_ref,
                   to_ref, so_ref):
  tf = tf_ref[...]
  sf = sf_ref[...]
  a0 = a0_ref[...]
  a1 = a1_ref[...]
  ht = jnp.maximum(
      jnp.dot(tf, w1tf_ref[...], preferred_element_type=jnp.float32)
      + jnp.dot(a0, w1ta_ref[...], preferred_element_type=jnp.float32)
      + b1t_ref[0:1, :], 0.0)
  to_ref[...] = tf + jnp.dot(ht, w2t_ref[...],
                             preferred_element_type=jnp.float32) + b2t_ref[0:1, :]
  hs = jnp.maximum(
      jnp.dot(sf, w1sf_ref[...], preferred_element_type=jnp.float32)
      + jnp.dot(a1, w1sa_ref[...], preferred_element_type=jnp.float32)
      + b1s_ref[0:1, :], 0.0)
  so_ref[...] = sf + jnp.dot(hs, w2s_ref[...],
                             preferred_element_type=jnp.float32) + b2s_ref[0:1, :]


def _tc_node_mlp(tf, sf, a0, a1, w1tf, w1ta, b1t, w2t, b2t,
                 w1sf, w1sa, b1s, w2s, b2s):
  grid = (N // BN,)
  full = lambda i: (0, 0)
  blk = lambda i: (i, 0)
  wspec = pl.BlockSpec((D, D), full)
  bspec = pl.BlockSpec((8, D), full)
  return pl.pallas_call(
      _node_mlp_body,
      grid=grid,
      in_specs=[
          pl.BlockSpec((BN, D), blk),
          pl.BlockSpec((BN, D), blk),
          pl.BlockSpec((BN, D), blk),
          pl.BlockSpec((BN, D), blk),
          wspec, wspec, bspec, wspec, bspec,
          wspec, wspec, bspec, wspec, bspec,
      ],
      out_specs=[
          pl.BlockSpec((BN, D), blk),
          pl.BlockSpec((BN, D), blk),
      ],
      out_shape=[
          jax.ShapeDtypeStruct((N, D), jnp.float32),
          jax.ShapeDtypeStruct((N, D), jnp.float32),
      ],
  )(tf, sf, a0, a1, w1tf, w1ta, b1t, w2t, b2t, w1sf, w1sa, b1s, w2s, b2s)


def kernel(src_node_feat, tgt_node_feat, src_node_coord, tgt_node_coord,
           edge_list, es_W1, es_b1, es_W2, es_b2, et_W1, et_b1, et_W2, et_b2,
           ns_W1, ns_b1, ns_W2, ns_b2, nt_W1, nt_b1, nt_W2, nt_b2):
  f32 = jnp.float32
  es = edge_list[0].astype(jnp.int32)
  et = edge_list[1].astype(jnp.int32)
  es2d = es.reshape(ROWS, 128)
  et2d = et.reshape(ROWS, 128)

  def coord_cols(c):
    return (c[:, 0], c[:, 1], c[:, 2])

  coords = coord_cols(src_node_coord) + coord_cols(tgt_node_coord)

  w1s = jnp.concatenate([es_W1[:, :D].T, et_W1[:, :D].T], axis=1)
  w1t = jnp.concatenate([es_W1[:, D:2 * D].T, et_W1[:, D:2 * D].T], axis=1)
  vb = jnp.tile(jnp.concatenate([es_W1[:, 2 * D], et_W1[:, 2 * D]])[None, :],
                (8, 1))
  bb = jnp.tile(jnp.concatenate([es_b1, et_b1])[None, :], (8, 1))
  t8 = lambda x: jnp.tile(x[None, :], (8, 1))
  mlp_w = (w1s, w1t, vb, bb, es_W2.T, t8(es_b2), et_W2.T, t8(et_b2))

  gather_main = _make_gather(CHUNK)
  gather_tail = _make_gather(TAIL)
  scatter_main = _make_scatter(CHUNK)
  scatter_tail = _make_scatter(TAIL)

  bounds = [(0, CHUNK, gather_main, scatter_main),
            (CHUNK, 2 * CHUNK, gather_main, scatter_main),
            (2 * CHUNK, ROWS, gather_tail, scatter_tail)]

  gathered = []
  for lo, hi, gfn, _ in bounds:
    gathered.append(gfn(src_node_feat, tgt_node_feat,
                        es2d[lo:hi], et2d[lo:hi], *coords))

  agg0 = jnp.zeros((N, D), f32)
  agg1 = jnp.zeros((N, D), f32)
  for (lo, hi, _, sfn), (xs, xt, rad) in zip(bounds, gathered):
    h2es, h2et = _tc_edge_mlp(xs, xt, rad, *mlp_w)
    agg0, agg1 = sfn(h2es, h2et, et2d[lo:hi], es2d[lo:hi], agg0, agg1)

  tgt_out, src_out = _tc_node_mlp(
      tgt_node_feat, src_node_feat, agg0, agg1,
      nt_W1[:, :D].T, nt_W1[:, D:].T, t8(nt_b1), nt_W2.T, t8(nt_b2),
      ns_W1[:, :D].T, ns_W1[:, D:].T, t8(ns_b1), ns_W2.T, t8(ns_b2))
  return (tgt_out, src_out)
